# Initial kernel scaffold; baseline (speedup 1.0000x reference)
import jax
import jax.numpy as jnp
from jax.experimental import pallas as pl


def kernel(sent_group_scores, sel_sent_emb, sel_sent_masks, group_embs, candi_sent_masks, bias):
    raise NotImplementedError("write your pallas kernel here")


# speedup vs baseline: 1.0014x; 1.0014x over previous
import jax
import jax.numpy as jnp
from jax.experimental import pallas as pl
from jax.experimental.pallas import tpu as pltpu


def _ewise_kernel(bias_ref, scores_ref, mask_ref, out_ref):
    out_ref[...] = (scores_ref[...] + bias_ref[0]) * mask_ref[...].astype(jnp.float32)


def kernel(sent_group_scores, sel_sent_emb, sel_sent_masks, group_embs, candi_sent_masks, bias):
    B, S = sent_group_scores.shape
    bias_arr = jnp.reshape(bias, (1,)).astype(jnp.float32)
    return pl.pallas_call(
        _ewise_kernel,
        out_shape=jax.ShapeDtypeStruct((B, S), jnp.float32),
        in_specs=[
            pl.BlockSpec(memory_space=pltpu.SMEM),
            pl.BlockSpec((B, S), lambda: (0, 0)),
            pl.BlockSpec((B, S), lambda: (0, 0)),
        ],
        out_specs=pl.BlockSpec((B, S), lambda: (0, 0)),
    )(bias_arr, sent_group_scores, candi_sent_masks)
